# baseline (speedup 1.0000x reference)
import functools

import jax
import jax.numpy as jnp
from jax import lax
from jax.experimental import pallas as pl
from jax.experimental.pallas import tpu as pltpu
from jax.experimental.pallas import tpu_sc as plsc

N = 16777
D = 128
K9 = 9
NC, NS, L = 2, 16, 16
NW = NC * NS
CH = 32
CPW = 17
RPW = CH * CPW
NPAD = NW * RPW
NCHUNK = NPAD // CH
CH1 = RPW // 4

_mesh = plsc.VectorSubcoreMesh(core_axis_name="c", subcore_axis_name="s")
_sc_params = pltpu.CompilerParams(needs_layout_passes=False)


def _sc_gather1_body(xp_hbm, gidx_hbm, out_hbm, x_v, idx_v, g_v, semx, semi, semo):
    wid = lax.axis_index("s") * NC + lax.axis_index("c")
    pltpu.async_copy(xp_hbm, x_v, semx).wait()
    handles = {}
    oh = {}
    for t in range(4):
        s = t % 2
        if t == 0:
            pltpu.sync_copy(gidx_hbm.at[wid, 0], idx_v.at[0])
        def body(m, _, s=s):
            iv = idx_v[s, pl.ds(m * L, L)]
            g_v[s, pl.ds(m * L, L)] = plsc.load_gather(x_v, [iv])
            return 0
        lax.fori_loop(0, (CH1 * 128) // L, body, 0)
        if t + 1 < 4:
            pltpu.sync_copy(gidx_hbm.at[wid, t + 1], idx_v.at[1 - s])
        if t >= 2:
            oh[s].wait()
        row0 = (wid * RPW + t * CH1) * 128
        oh[s] = pltpu.async_copy(g_v.at[s], out_hbm.at[pl.ds(row0, CH1 * 128)], semo.at[s])
    oh[0].wait()
    oh[1].wait()


@functools.partial(jax.jit, static_argnums=())
def _sc_gather1(xp, gidx):
    fn = pl.kernel(
        _sc_gather1_body,
        mesh=_mesh,
        out_type=jax.ShapeDtypeStruct((NPAD * 128,), jnp.float32),
        scratch_types=[
            pltpu.VMEM((NPAD,), jnp.float32),
            pltpu.VMEM((2, CH1 * 128), jnp.int32),
            pltpu.VMEM((2, CH1 * 128), jnp.float32),
            pltpu.SemaphoreType.DMA,
            pltpu.SemaphoreType.DMA,
            pltpu.SemaphoreType.DMA((2,)),
        ],
        compiler_params=_sc_params,
    )
    return fn(xp, gidx)


def _sc_conv_body(ztab_hbm, gidx_hbm, out_hbm, idx_v, g_v, acc_v, semg, semo):
    wid = lax.axis_index("s") * NC + lax.axis_index("c")
    base = wid * CPW
    gh = {}
    oh = {}

    def fire(t, s):
        pltpu.sync_copy(gidx_hbm.at[base + t], idx_v.at[s])
        gh[s] = [
            pltpu.async_copy(ztab_hbm.at[idx_v.at[s, k]], g_v.at[s, k], semg.at[s])
            for k in range(K9)
        ]

    fire(0, 0)
    for t in range(CPW):
        s = t % 2
        if t + 1 < CPW:
            fire(t + 1, 1 - s)
        for h in gh[s]:
            h.wait()

        def body(r, _, s=s):
            for j in range(D // L):
                a = g_v[s, 0, r, pl.ds(j * L, L)]
                for k in range(1, K9):
                    a = a + g_v[s, k, r, pl.ds(j * L, L)]
                acc_v[s, r, pl.ds(j * L, L)] = a
            return 0
        if t >= 2:
            oh[s].wait()
        lax.fori_loop(0, CH, body, 0)
        oh[s] = pltpu.async_copy(
            acc_v.at[s], out_hbm.at[pl.ds((base + t) * CH, CH)], semo.at[s]
        )
    oh[(CPW - 1) % 2].wait()
    oh[CPW % 2].wait()


def _sc_conv(ztab, gidx):
    fn = pl.kernel(
        _sc_conv_body,
        mesh=_mesh,
        out_type=jax.ShapeDtypeStruct((NPAD, D), jnp.float32),
        scratch_types=[
            pltpu.VMEM((2, K9, CH), jnp.int32),
            pltpu.VMEM((2, K9, CH, D), jnp.float32),
            pltpu.VMEM((2, CH, D), jnp.float32),
            pltpu.SemaphoreType.DMA((2,)),
            pltpu.SemaphoreType.DMA((2,)),
        ],
        compiler_params=_sc_params,
    )
    return fn(ztab, gidx)


def _mm_body(a_ref, w_ref, o_ref, *, relu_in):
    a = a_ref[...]
    if relu_in:
        a = jnp.maximum(a, 0.0)
    o_ref[...] = jnp.dot(a, w_ref[...], preferred_element_type=jnp.float32)


def _tc_matmul(a, w, relu_in, bm=1024):
    m, kin = a.shape
    kout = w.shape[1]
    return pl.pallas_call(
        functools.partial(_mm_body, relu_in=relu_in),
        grid=(m // bm,),
        in_specs=[
            pl.BlockSpec((bm, kin), lambda i: (i, 0)),
            pl.BlockSpec((kin, kout), lambda i: (0, 0)),
        ],
        out_specs=pl.BlockSpec((bm, kout), lambda i: (i, 0)),
        out_shape=jax.ShapeDtypeStruct((m, kout), jnp.float32),
    )(a, w)


def _head_body(s1, s2, s3, s4, wf1, bf1, wf2, bf2, o_ref, acc):
    i = pl.program_id(0)

    @pl.when(i == 0)
    def _init():
        acc[...] = jnp.zeros_like(acc)

    for idx, s in enumerate((s1, s2, s3, s4)):
        acc[0:1, idx * D:(idx + 1) * D] += jnp.sum(
            jnp.maximum(s[...], 0.0), axis=0, keepdims=True
        )

    @pl.when(i == pl.num_programs(0) - 1)
    def _final():
        p = acc[...] * (1.0 / N)
        h = jnp.maximum(
            jnp.dot(p, wf1[...], preferred_element_type=jnp.float32) + bf1[...], 0.0
        )
        o_ref[...] = jnp.dot(h, wf2[...], preferred_element_type=jnp.float32) + bf2[...]


def _tc_head(s1, s2, s3, s4, wf1, bf1, wf2, bf2, bm=1024):
    g = NPAD // bm
    sspec = pl.BlockSpec((bm, D), lambda i: (i, 0))
    full = lambda shape: pl.BlockSpec(shape, lambda i: (0, 0))
    return pl.pallas_call(
        _head_body,
        grid=(g,),
        in_specs=[sspec, sspec, sspec, sspec,
                  full((512, 512)), full((1, 512)), full((512, D)), full((1, D))],
        out_specs=full((1, D)),
        out_shape=jax.ShapeDtypeStruct((1, D), jnp.float32),
        scratch_shapes=[pltpu.VMEM((1, 512), jnp.float32)],
    )(s1, s2, s3, s4, wf1, bf1, wf2, bf2)


def _conv_gidx(neigh):
    np_ = jnp.full((NPAD, K9), N, jnp.int32).at[:N].set(neigh.astype(jnp.int32))
    g = np_ * K9 + jnp.arange(K9, dtype=jnp.int32)[None, :]
    return g.reshape(NCHUNK, CH, K9).transpose(0, 2, 1)


def kernel(x, neigh5, neigh3d1, neigh3d2, neigh3d3, W5, W2a, W2b, W2c,
           W3a, W3b, W3c, W4a, W4b, W4c, Wf1, bf1, Wf2, bf2):
    xp = jnp.zeros((NPAD,), jnp.float32).at[:N].set(x[:, 0])
    g1 = jnp.full((NPAD, 128), N, jnp.int32).at[:N, :25].set(neigh5.astype(jnp.int32))
    g1 = g1.reshape(NW, 4, CH1 * 128)
    W5p = jnp.zeros((128, 128), jnp.float32).at[:25].set(W5[:, 0, :])

    G = _sc_gather1(xp, g1).reshape(NPAD, 128)
    S1 = _tc_matmul(G, W5p, relu_in=False)

    gmaps = [_conv_gidx(n) for n in (neigh3d1, neigh3d2, neigh3d3)]
    Ws = [W2a, W2b, W2c, W3a, W3b, W3c, W4a, W4b, W4c]
    cur = S1
    pooled = [S1]
    for li in range(9):
        Wcat = Ws[li].transpose(1, 0, 2).reshape(D, K9 * D)
        Z = _tc_matmul(cur, Wcat, relu_in=True)
        cur = _sc_conv(Z.reshape(NPAD * K9, D), gmaps[li % 3])
        if li in (2, 5, 8):
            pooled.append(cur)

    return _tc_head(pooled[0], pooled[1], pooled[2], pooled[3],
                    Wf1, bf1.reshape(1, 512), Wf2, bf2.reshape(1, D))
